# Initial kernel scaffold; baseline (speedup 1.0000x reference)
import jax
import jax.numpy as jnp
from jax.experimental import pallas as pl


def kernel(x, edge_index, W1, b1, W2, b2, W3, b3):
    raise NotImplementedError("write your pallas kernel here")


# speedup vs baseline: 5.8556x; 5.8556x over previous
import functools
import jax
import jax.numpy as jnp
from jax import lax
from jax.experimental import pallas as pl
from jax.experimental.pallas import tpu as pltpu
from jax.experimental.pallas import tpu_sc as plsc

N = 10000
E = 160000
NC = 2
NS = 16
STRIPE = 624
LAST = N - (NS - 1) * STRIPE
CH = 80
DCH = 40
TN = 1000


def _mesh():
    return plsc.VectorSubcoreMesh(
        core_axis_name="c", subcore_axis_name="s", num_cores=NC, num_subcores=NS
    )


def _per_stripe(s, do):
    @pl.when(s < NS - 1)
    def _():
        do(s * STRIPE, STRIPE)

    @pl.when(s == NS - 1)
    def _():
        do((NS - 1) * STRIPE, LAST)


@functools.cache
def _make_deg():
    return functools.partial(
        pl.kernel,
        out_type=jax.ShapeDtypeStruct((NC, N, 128), jnp.float32),
        mesh=_mesh(),
        scratch_types=[
            pltpu.VMEM_SHARED((N, 128), jnp.float32),
            pltpu.VMEM((DCH,), jnp.int32),
            pltpu.VMEM((DCH, 128), jnp.float32),
        ],
    )(_deg_body)


def _deg_body(dst_ref, zeros_ref, ones_ref, out_ref, acc, idx_v, ones_v):
    c = lax.axis_index("c")
    s = lax.axis_index("s")
    _per_stripe(s, lambda st, sz: pltpu.sync_copy(
        zeros_ref.at[pl.ds(0, sz)], acc.at[pl.ds(st, sz)]))
    pltpu.sync_copy(ones_ref, ones_v)
    plsc.subcore_barrier()
    per_tile = E // (NC * NS)
    ebase = (c * NS + s) * per_tile

    def chunk(k, carry):
        pltpu.sync_copy(dst_ref.at[pl.ds(ebase + k * DCH, DCH)], idx_v)
        pltpu.sync_copy(ones_v, acc.at[idx_v], add=True)
        return carry

    lax.fori_loop(0, per_tile // DCH, chunk, 0)
    plsc.subcore_barrier()

    @pl.when(c == 0)
    def _():
        _per_stripe(s, lambda st, sz: pltpu.sync_copy(
            acc.at[pl.ds(st, sz)], out_ref.at[0, pl.ds(st, sz)]))

    @pl.when(c == 1)
    def _():
        _per_stripe(s, lambda st, sz: pltpu.sync_copy(
            acc.at[pl.ds(st, sz)], out_ref.at[1, pl.ds(st, sz)]))


@functools.cache
def _make_prop(S):
    per_tile = E // NS

    @functools.partial(
        pl.kernel,
        out_type=jax.ShapeDtypeStruct((S, N, 128), jnp.float32),
        mesh=_mesh(),
        scratch_types=[
            pltpu.VMEM_SHARED((N, 128), jnp.float32),
            pltpu.VMEM((CH,), jnp.int32),
            pltpu.VMEM((CH,), jnp.int32),
            pltpu.VMEM((CH, 128), jnp.float32),
            pltpu.SemaphoreType.DMA,
        ],
    )
    def prop(z_ref, src_ref, dst_ref, out_ref, acc, sidx, didx, rows, sem):
        c = lax.axis_index("c")
        s = lax.axis_index("s")
        ebase = s * per_tile
        for slab in range(S):

            @pl.when(c == (slab % NC))
            def _(slab=slab):
                _per_stripe(s, lambda st, sz: pltpu.sync_copy(
                    z_ref.at[slab, pl.ds(st, sz)], acc.at[pl.ds(st, sz)]))
                plsc.subcore_barrier()

                def chunk(k, carry):
                    off = ebase + k * CH
                    pltpu.sync_copy(src_ref.at[pl.ds(off, CH)], sidx)
                    pltpu.sync_copy(dst_ref.at[pl.ds(off, CH)], didx)
                    pltpu.async_copy(z_ref.at[slab].at[sidx], rows, sem).wait()
                    pltpu.sync_copy(rows, acc.at[didx], add=True)
                    return carry

                lax.fori_loop(0, per_tile // CH, chunk, 0)
                plsc.subcore_barrier()
                _per_stripe(s, lambda st, sz: pltpu.sync_copy(
                    acc.at[pl.ds(st, sz)], out_ref.at[slab, pl.ds(st, sz)]))

    return prop


def _dinv_block(deg_ref):
    d = deg_ref[0][:, 0:1] + deg_ref[1][:, 0:1] + 1.0
    return lax.rsqrt(d)


def _mm1_body(deg_ref, x_ref, w_ref, out_ref):
    dinv = _dinv_block(deg_ref)
    out_ref[...] = jnp.dot(
        x_ref[...], w_ref[...], preferred_element_type=jnp.float32
    ) * dinv


def _mm1(deg, x, W):
    S = W.shape[1] // 128
    return pl.pallas_call(
        _mm1_body,
        grid=(N // TN, S),
        in_specs=[
            pl.BlockSpec((NC, TN, 128), lambda n, t: (0, n, 0)),
            pl.BlockSpec((TN, x.shape[1]), lambda n, t: (n, 0)),
            pl.BlockSpec((W.shape[0], 128), lambda n, t: (0, t)),
        ],
        out_specs=pl.BlockSpec((None, TN, 128), lambda n, t: (t, n, 0)),
        out_shape=jax.ShapeDtypeStruct((S, N, 128), jnp.float32),
    )(deg, x, W)


def _make_mid(S_in):
    def body(deg_ref, agg_ref, w_ref, b_ref, out_ref):
        dinv = _dinv_block(deg_ref)
        hs = [
            jnp.maximum(agg_ref[si] * dinv + b_ref[si], 0.0) for si in range(S_in)
        ]
        h = jnp.concatenate(hs, axis=1)
        out_ref[...] = jnp.dot(
            h, w_ref[...], preferred_element_type=jnp.float32
        ) * dinv

    def run(deg, agg, W, b):
        S_out = W.shape[1] // 128
        return pl.pallas_call(
            body,
            grid=(N // TN, S_out),
            in_specs=[
                pl.BlockSpec((NC, TN, 128), lambda n, t: (0, n, 0)),
                pl.BlockSpec((S_in, TN, 128), lambda n, t: (0, n, 0)),
                pl.BlockSpec((W.shape[0], 128), lambda n, t: (0, t)),
                pl.BlockSpec((S_in, 1, 128), lambda n, t: (0, 0, 0)),
            ],
            out_specs=pl.BlockSpec((None, TN, 128), lambda n, t: (t, n, 0)),
            out_shape=jax.ShapeDtypeStruct((S_out, N, 128), jnp.float32),
        )(deg, agg, W, b.reshape(S_in, 1, 128))

    return run


_mm_mid = _make_mid(4)


def _fin_body(deg_ref, agg_ref, b_ref, h_ref, ls_ref):
    dinv = _dinv_block(deg_ref)
    h = jnp.concatenate(
        [agg_ref[0] * dinv + b_ref[0], agg_ref[1] * dinv + b_ref[1]], axis=1
    )
    h_ref[...] = h
    m = jnp.max(h, axis=1, keepdims=True)
    lse = jnp.log(jnp.sum(jnp.exp(h - m), axis=1, keepdims=True)) + m
    ls_ref[...] = h - lse


def _fin(deg, agg, b):
    return pl.pallas_call(
        _fin_body,
        grid=(N // TN,),
        in_specs=[
            pl.BlockSpec((NC, TN, 128), lambda n: (0, n, 0)),
            pl.BlockSpec((2, TN, 128), lambda n: (0, n, 0)),
            pl.BlockSpec((2, 1, 128), lambda n: (0, 0, 0)),
        ],
        out_specs=[
            pl.BlockSpec((TN, 256), lambda n: (n, 0)),
            pl.BlockSpec((TN, 256), lambda n: (n, 0)),
        ],
        out_shape=[
            jax.ShapeDtypeStruct((N, 256), jnp.float32),
            jax.ShapeDtypeStruct((N, 256), jnp.float32),
        ],
    )(deg, agg, b.reshape(2, 1, 128))


@jax.jit
def kernel(x, edge_index, W1, b1, W2, b2, W3, b3):
    ei = edge_index.astype(jnp.int32)
    src = ei[0]
    dst = ei[1]
    zeros16 = jnp.zeros((LAST, 128), jnp.float32)
    ones16 = jnp.ones((DCH, 128), jnp.float32)
    deg = _make_deg()(dst, zeros16, ones16)

    z1 = _mm1(deg, x, W1)
    agg1 = _make_prop(4)(z1, src, dst)
    z2 = _mm_mid(deg, agg1, W2, b1)
    agg2 = _make_prop(4)(z2, src, dst)
    z3 = _mm_mid(deg, agg2, W3, b2)
    agg3 = _make_prop(2)(z3, src, dst)
    h, ls = _fin(deg, agg3, b3)
    return (h, ls)


# speedup vs baseline: 10.3713x; 1.7712x over previous
import functools
import jax
import jax.numpy as jnp
from jax import lax
from jax.experimental import pallas as pl
from jax.experimental.pallas import tpu as pltpu
from jax.experimental.pallas import tpu_sc as plsc

N = 10000
E = 160000
NC = 2
NS = 16
STRIPE = 624
LAST = N - (NS - 1) * STRIPE
CH = 40
DCH = 40
TN = 1000


def _mesh():
    return plsc.VectorSubcoreMesh(
        core_axis_name="c", subcore_axis_name="s", num_cores=NC, num_subcores=NS
    )


def _per_stripe(s, do):
    @pl.when(s < NS - 1)
    def _():
        do(s * STRIPE, STRIPE)

    @pl.when(s == NS - 1)
    def _():
        do((NS - 1) * STRIPE, LAST)


NBUF = 5

@functools.cache
def _make_deg():
    nch = E // (NC * NS) // DCH

    @functools.partial(
        pl.kernel,
        out_type=jax.ShapeDtypeStruct((NC, N, 128), jnp.float32),
        mesh=_mesh(),
        scratch_types=[
            pltpu.VMEM_SHARED((N, 128), jnp.float32),
            pltpu.VMEM((nch, DCH), jnp.int32),
            pltpu.VMEM((DCH, 128), jnp.float32),
            pltpu.SemaphoreType.DMA((NBUF,)),
        ],
    )
    def deg(dst3_ref, zeros_ref, ones_ref, out_ref, acc, didx, ones_v, ssem):
        c = lax.axis_index("c")
        s = lax.axis_index("s")
        _per_stripe(s, lambda st, sz: pltpu.sync_copy(
            zeros_ref.at[pl.ds(0, sz)], acc.at[pl.ds(st, sz)]))
        pltpu.sync_copy(ones_ref, ones_v)
        pltpu.sync_copy(dst3_ref.at[c * NS + s], didx)
        plsc.subcore_barrier()

        def group(g, carry):
            descs = []
            for b in range(NBUF):
                k = g * NBUF + b
                descs.append(pltpu.async_copy(
                    ones_v, acc.at[didx.at[k]], ssem.at[b], add=True))
            for d in descs:
                d.wait()
            return carry

        lax.fori_loop(0, nch // NBUF, group, 0)
        plsc.subcore_barrier()

        @pl.when(c == 0)
        def _():
            _per_stripe(s, lambda st, sz: pltpu.sync_copy(
                acc.at[pl.ds(st, sz)], out_ref.at[0, pl.ds(st, sz)]))

        @pl.when(c == 1)
        def _():
            _per_stripe(s, lambda st, sz: pltpu.sync_copy(
                acc.at[pl.ds(st, sz)], out_ref.at[1, pl.ds(st, sz)]))

    return deg


@functools.cache
def _make_prop(S):
    per_tile = E // NS
    nch = per_tile // CH

    @functools.partial(
        pl.kernel,
        out_type=jax.ShapeDtypeStruct((S, N, 128), jnp.float32),
        mesh=_mesh(),
        scratch_types=[
            pltpu.VMEM_SHARED((N, 128), jnp.float32),
            pltpu.VMEM((NBUF, CH), jnp.int32),
            pltpu.VMEM((NBUF, CH), jnp.int32),
            pltpu.VMEM((NBUF, CH, 128), jnp.float32),
            pltpu.SemaphoreType.DMA((NBUF,)),
            pltpu.SemaphoreType.DMA((NBUF,)),
            pltpu.SemaphoreType.DMA((NBUF,)),
            pltpu.SemaphoreType.DMA((NBUF,)),
        ],
    )
    def prop(z_ref, src_ref, dst_ref, out_ref, acc, sidx, didx, rows,
             isem, dsem, gsem, ssem):
        c = lax.axis_index("c")
        s = lax.axis_index("s")
        ebase = s * per_tile
        for slab in range(S):

            @pl.when(c == (slab % NC))
            def _(slab=slab):
                _per_stripe(s, lambda st, sz: pltpu.sync_copy(
                    z_ref.at[slab, pl.ds(st, sz)], acc.at[pl.ds(st, sz)]))
                plsc.subcore_barrier()

                def group(g, carry):
                    ids, dds, gds, sds = [], [], [], []
                    for b in range(NBUF):
                        off = ebase + (g * NBUF + b) * CH
                        ids.append(pltpu.async_copy(
                            src_ref.at[pl.ds(off, CH)], sidx.at[b], isem.at[b]))
                        dds.append(pltpu.async_copy(
                            dst_ref.at[pl.ds(off, CH)], didx.at[b], dsem.at[b]))
                    for b in range(NBUF):
                        ids[b].wait()
                        gds.append(pltpu.async_copy(
                            z_ref.at[slab].at[sidx.at[b]], rows.at[b], gsem.at[b]))
                    for b in range(NBUF):
                        dds[b].wait()
                        gds[b].wait()
                        sds.append(pltpu.async_copy(
                            rows.at[b], acc.at[didx.at[b]], ssem.at[b], add=True))
                    for d in sds:
                        d.wait()
                    return carry

                lax.fori_loop(0, nch // NBUF, group, 0)
                plsc.subcore_barrier()
                _per_stripe(s, lambda st, sz: pltpu.sync_copy(
                    acc.at[pl.ds(st, sz)], out_ref.at[slab, pl.ds(st, sz)]))

    return prop


def _dinv_block(deg_ref):
    d = deg_ref[0][:, 0:1] + deg_ref[1][:, 0:1] + 1.0
    return lax.rsqrt(d)


def _mm1_body(deg_ref, x_ref, w_ref, out_ref):
    dinv = _dinv_block(deg_ref)
    out_ref[...] = jnp.dot(
        x_ref[...], w_ref[...], preferred_element_type=jnp.float32
    ) * dinv


def _mm1(deg, x, W):
    S = W.shape[1] // 128
    return pl.pallas_call(
        _mm1_body,
        grid=(N // TN, S),
        in_specs=[
            pl.BlockSpec((NC, TN, 128), lambda n, t: (0, n, 0)),
            pl.BlockSpec((TN, x.shape[1]), lambda n, t: (n, 0)),
            pl.BlockSpec((W.shape[0], 128), lambda n, t: (0, t)),
        ],
        out_specs=pl.BlockSpec((None, TN, 128), lambda n, t: (t, n, 0)),
        out_shape=jax.ShapeDtypeStruct((S, N, 128), jnp.float32),
    )(deg, x, W)


def _make_mid(S_in):
    def body(deg_ref, agg_ref, w_ref, b_ref, out_ref):
        dinv = _dinv_block(deg_ref)
        hs = [
            jnp.maximum(agg_ref[si] * dinv + b_ref[si], 0.0) for si in range(S_in)
        ]
        h = jnp.concatenate(hs, axis=1)
        out_ref[...] = jnp.dot(
            h, w_ref[...], preferred_element_type=jnp.float32
        ) * dinv

    def run(deg, agg, W, b):
        S_out = W.shape[1] // 128
        return pl.pallas_call(
            body,
            grid=(N // TN, S_out),
            in_specs=[
                pl.BlockSpec((NC, TN, 128), lambda n, t: (0, n, 0)),
                pl.BlockSpec((S_in, TN, 128), lambda n, t: (0, n, 0)),
                pl.BlockSpec((W.shape[0], 128), lambda n, t: (0, t)),
                pl.BlockSpec((S_in, 1, 128), lambda n, t: (0, 0, 0)),
            ],
            out_specs=pl.BlockSpec((None, TN, 128), lambda n, t: (t, n, 0)),
            out_shape=jax.ShapeDtypeStruct((S_out, N, 128), jnp.float32),
        )(deg, agg, W, b.reshape(S_in, 1, 128))

    return run


_mm_mid = _make_mid(4)


def _fin_body(deg_ref, agg_ref, b_ref, h_ref, ls_ref):
    dinv = _dinv_block(deg_ref)
    h = jnp.concatenate(
        [agg_ref[0] * dinv + b_ref[0], agg_ref[1] * dinv + b_ref[1]], axis=1
    )
    h_ref[...] = h
    m = jnp.max(h, axis=1, keepdims=True)
    lse = jnp.log(jnp.sum(jnp.exp(h - m), axis=1, keepdims=True)) + m
    ls_ref[...] = h - lse


def _fin(deg, agg, b):
    return pl.pallas_call(
        _fin_body,
        grid=(N // TN,),
        in_specs=[
            pl.BlockSpec((NC, TN, 128), lambda n: (0, n, 0)),
            pl.BlockSpec((2, TN, 128), lambda n: (0, n, 0)),
            pl.BlockSpec((2, 1, 128), lambda n: (0, 0, 0)),
        ],
        out_specs=[
            pl.BlockSpec((TN, 256), lambda n: (n, 0)),
            pl.BlockSpec((TN, 256), lambda n: (n, 0)),
        ],
        out_shape=[
            jax.ShapeDtypeStruct((N, 256), jnp.float32),
            jax.ShapeDtypeStruct((N, 256), jnp.float32),
        ],
    )(deg, agg, b.reshape(2, 1, 128))


@jax.jit
def kernel(x, edge_index, W1, b1, W2, b2, W3, b3):
    ei = edge_index.astype(jnp.int32)
    src = ei[0]
    dst = ei[1]
    dst3d = dst.reshape(NC * NS, -1, DCH)
    zeros16 = jnp.zeros((LAST, 128), jnp.float32)
    ones16 = jnp.ones((DCH, 128), jnp.float32)
    deg = _make_deg()(dst3d, zeros16, ones16)

    z1 = _mm1(deg, x, W1)
    agg1 = _make_prop(4)(z1, src, dst)
    z2 = _mm_mid(deg, agg1, W2, b1)
    agg2 = _make_prop(4)(z2, src, dst)
    z3 = _mm_mid(deg, agg2, W3, b2)
    agg3 = _make_prop(2)(z3, src, dst)
    h, ls = _fin(deg, agg3, b3)
    return (h, ls)


# speedup vs baseline: 11.6550x; 1.1238x over previous
import functools
import jax
import jax.numpy as jnp
from jax import lax
from jax.experimental import pallas as pl
from jax.experimental.pallas import tpu as pltpu
from jax.experimental.pallas import tpu_sc as plsc

N = 10000
E = 160000
NC = 2
NS = 16
STRIPE = 624
LAST = N - (NS - 1) * STRIPE
CH = 40
DCH = 40
TN = 1000


def _mesh():
    return plsc.VectorSubcoreMesh(
        core_axis_name="c", subcore_axis_name="s", num_cores=NC, num_subcores=NS
    )


def _per_stripe(s, do):
    @pl.when(s < NS - 1)
    def _():
        do(s * STRIPE, STRIPE)

    @pl.when(s == NS - 1)
    def _():
        do((NS - 1) * STRIPE, LAST)


NBUF = 5

@functools.cache
def _make_deg():
    nch = E // (NC * NS) // DCH

    @functools.partial(
        pl.kernel,
        out_type=jax.ShapeDtypeStruct((NC, N, 128), jnp.float32),
        mesh=_mesh(),
        scratch_types=[
            pltpu.VMEM_SHARED((N, 128), jnp.float32),
            pltpu.VMEM((nch, DCH), jnp.int32),
            pltpu.VMEM((DCH, 128), jnp.float32),
            pltpu.SemaphoreType.DMA((NBUF,)),
        ],
    )
    def deg(dst3_ref, zeros_ref, ones_ref, out_ref, acc, didx, ones_v, ssem):
        c = lax.axis_index("c")
        s = lax.axis_index("s")
        _per_stripe(s, lambda st, sz: pltpu.sync_copy(
            zeros_ref.at[pl.ds(0, sz)], acc.at[pl.ds(st, sz)]))
        pltpu.sync_copy(ones_ref, ones_v)
        pltpu.sync_copy(dst3_ref.at[c * NS + s], didx)
        plsc.subcore_barrier()

        def chunk(k, carry):
            pltpu.async_copy(ones_v, acc.at[didx.at[k]], ssem.at[0], add=True)
            return carry

        lax.fori_loop(0, nch, chunk, 0)

        def drain(k, carry):
            pltpu.make_async_copy(ones_v, acc.at[didx.at[k]], ssem.at[0]).wait()
            return carry

        lax.fori_loop(0, nch, drain, 0)
        plsc.subcore_barrier()

        @pl.when(c == 0)
        def _():
            _per_stripe(s, lambda st, sz: pltpu.sync_copy(
                acc.at[pl.ds(st, sz)], out_ref.at[0, pl.ds(st, sz)]))

        @pl.when(c == 1)
        def _():
            _per_stripe(s, lambda st, sz: pltpu.sync_copy(
                acc.at[pl.ds(st, sz)], out_ref.at[1, pl.ds(st, sz)]))

    return deg


@functools.cache
def _make_prop(S):
    per_tile = E // NS
    nch = per_tile // CH

    @functools.partial(
        pl.kernel,
        out_type=jax.ShapeDtypeStruct((S, N, 128), jnp.float32),
        mesh=_mesh(),
        scratch_types=[
            pltpu.VMEM_SHARED((N, 128), jnp.float32),
            pltpu.VMEM((NBUF, CH), jnp.int32),
            pltpu.VMEM((NBUF, CH), jnp.int32),
            pltpu.VMEM((NBUF, CH, 128), jnp.float32),
            pltpu.SemaphoreType.DMA((NBUF,)),
            pltpu.SemaphoreType.DMA((NBUF,)),
            pltpu.SemaphoreType.DMA((NBUF,)),
            pltpu.SemaphoreType.DMA((NBUF,)),
        ],
    )
    def prop(z_ref, src_ref, dst_ref, out_ref, acc, sidx, didx, rows,
             isem, dsem, gsem, ssem):
        c = lax.axis_index("c")
        s = lax.axis_index("s")
        ebase = s * per_tile
        for slab in range(S):

            @pl.when(c == (slab % NC))
            def _(slab=slab):
                _per_stripe(s, lambda st, sz: pltpu.sync_copy(
                    z_ref.at[slab, pl.ds(st, sz)], acc.at[pl.ds(st, sz)]))
                plsc.subcore_barrier()

                def group(g, carry):
                    ids, dds, gds = [], [], []
                    for b in range(NBUF):
                        off = ebase + (g * NBUF + b) * CH

                        @pl.when(g > 0)
                        def _(b=b):
                            pltpu.make_async_copy(
                                rows.at[b], acc.at[didx.at[b]], ssem.at[b]
                            ).wait()

                        ids.append(pltpu.async_copy(
                            src_ref.at[pl.ds(off, CH)], sidx.at[b], isem.at[b]))
                        dds.append(pltpu.async_copy(
                            dst_ref.at[pl.ds(off, CH)], didx.at[b], dsem.at[b]))
                    for b in range(NBUF):
                        ids[b].wait()
                        gds.append(pltpu.async_copy(
                            z_ref.at[slab].at[sidx.at[b]], rows.at[b], gsem.at[b]))
                    for b in range(NBUF):
                        dds[b].wait()
                        gds[b].wait()
                        pltpu.async_copy(
                            rows.at[b], acc.at[didx.at[b]], ssem.at[b], add=True)
                    return carry

                lax.fori_loop(0, nch // NBUF, group, 0)
                for b in range(NBUF):
                    pltpu.make_async_copy(
                        rows.at[b], acc.at[didx.at[b]], ssem.at[b]).wait()
                plsc.subcore_barrier()
                _per_stripe(s, lambda st, sz: pltpu.sync_copy(
                    acc.at[pl.ds(st, sz)], out_ref.at[slab, pl.ds(st, sz)]))

    return prop


def _dinv_block(deg_ref):
    d = deg_ref[0][:, 0:1] + deg_ref[1][:, 0:1] + 1.0
    return lax.rsqrt(d)


def _mm1_body(deg_ref, x_ref, w_ref, out_ref):
    dinv = _dinv_block(deg_ref)
    out_ref[...] = jnp.dot(
        x_ref[...], w_ref[...], preferred_element_type=jnp.float32
    ) * dinv


def _mm1(deg, x, W):
    S = W.shape[1] // 128
    return pl.pallas_call(
        _mm1_body,
        grid=(N // TN, S),
        in_specs=[
            pl.BlockSpec((NC, TN, 128), lambda n, t: (0, n, 0)),
            pl.BlockSpec((TN, x.shape[1]), lambda n, t: (n, 0)),
            pl.BlockSpec((W.shape[0], 128), lambda n, t: (0, t)),
        ],
        out_specs=pl.BlockSpec((None, TN, 128), lambda n, t: (t, n, 0)),
        out_shape=jax.ShapeDtypeStruct((S, N, 128), jnp.float32),
    )(deg, x, W)


def _make_mid(S_in):
    def body(deg_ref, agg_ref, w_ref, b_ref, out_ref):
        dinv = _dinv_block(deg_ref)
        hs = [
            jnp.maximum(agg_ref[si] * dinv + b_ref[si], 0.0) for si in range(S_in)
        ]
        h = jnp.concatenate(hs, axis=1)
        out_ref[...] = jnp.dot(
            h, w_ref[...], preferred_element_type=jnp.float32
        ) * dinv

    def run(deg, agg, W, b):
        S_out = W.shape[1] // 128
        return pl.pallas_call(
            body,
            grid=(N // TN, S_out),
            in_specs=[
                pl.BlockSpec((NC, TN, 128), lambda n, t: (0, n, 0)),
                pl.BlockSpec((S_in, TN, 128), lambda n, t: (0, n, 0)),
                pl.BlockSpec((W.shape[0], 128), lambda n, t: (0, t)),
                pl.BlockSpec((S_in, 1, 128), lambda n, t: (0, 0, 0)),
            ],
            out_specs=pl.BlockSpec((None, TN, 128), lambda n, t: (t, n, 0)),
            out_shape=jax.ShapeDtypeStruct((S_out, N, 128), jnp.float32),
        )(deg, agg, W, b.reshape(S_in, 1, 128))

    return run


_mm_mid = _make_mid(4)


def _fin_body(deg_ref, agg_ref, b_ref, h_ref, ls_ref):
    dinv = _dinv_block(deg_ref)
    h = jnp.concatenate(
        [agg_ref[0] * dinv + b_ref[0], agg_ref[1] * dinv + b_ref[1]], axis=1
    )
    h_ref[...] = h
    m = jnp.max(h, axis=1, keepdims=True)
    lse = jnp.log(jnp.sum(jnp.exp(h - m), axis=1, keepdims=True)) + m
    ls_ref[...] = h - lse


def _fin(deg, agg, b):
    return pl.pallas_call(
        _fin_body,
        grid=(N // TN,),
        in_specs=[
            pl.BlockSpec((NC, TN, 128), lambda n: (0, n, 0)),
            pl.BlockSpec((2, TN, 128), lambda n: (0, n, 0)),
            pl.BlockSpec((2, 1, 128), lambda n: (0, 0, 0)),
        ],
        out_specs=[
            pl.BlockSpec((TN, 256), lambda n: (n, 0)),
            pl.BlockSpec((TN, 256), lambda n: (n, 0)),
        ],
        out_shape=[
            jax.ShapeDtypeStruct((N, 256), jnp.float32),
            jax.ShapeDtypeStruct((N, 256), jnp.float32),
        ],
    )(deg, agg, b.reshape(2, 1, 128))


@jax.jit
def kernel(x, edge_index, W1, b1, W2, b2, W3, b3):
    ei = edge_index.astype(jnp.int32)
    src = ei[0]
    dst = ei[1]
    dst3d = dst.reshape(NC * NS, -1, DCH)
    zeros16 = jnp.zeros((LAST, 128), jnp.float32)
    ones16 = jnp.ones((DCH, 128), jnp.float32)
    deg = _make_deg()(dst3d, zeros16, ones16)

    z1 = _mm1(deg, x, W1)
    agg1 = _make_prop(4)(z1, src, dst)
    z2 = _mm_mid(deg, agg1, W2, b1)
    agg2 = _make_prop(4)(z2, src, dst)
    z3 = _mm_mid(deg, agg2, W3, b2)
    agg3 = _make_prop(2)(z3, src, dst)
    h, ls = _fin(deg, agg3, b3)
    return (h, ls)


# speedup vs baseline: 11.9807x; 1.0279x over previous
import functools
import jax
import jax.numpy as jnp
from jax import lax
from jax.experimental import pallas as pl
from jax.experimental.pallas import tpu as pltpu
from jax.experimental.pallas import tpu_sc as plsc

N = 10000
E = 160000
NC = 2
NS = 16
STRIPE = 624
LAST = N - (NS - 1) * STRIPE
CH = 80
PBUF = 4
DCH = 40
TN = 1000


def _mesh():
    return plsc.VectorSubcoreMesh(
        core_axis_name="c", subcore_axis_name="s", num_cores=NC, num_subcores=NS
    )


def _per_stripe(s, do):
    @pl.when(s < NS - 1)
    def _():
        do(s * STRIPE, STRIPE)

    @pl.when(s == NS - 1)
    def _():
        do((NS - 1) * STRIPE, LAST)


NBUF = 5

@functools.cache
def _make_deg():
    nch = E // (NC * NS) // DCH

    @functools.partial(
        pl.kernel,
        out_type=jax.ShapeDtypeStruct((NC, N, 128), jnp.float32),
        mesh=_mesh(),
        scratch_types=[
            pltpu.VMEM_SHARED((N, 128), jnp.float32),
            pltpu.VMEM((nch, DCH), jnp.int32),
            pltpu.VMEM((DCH, 128), jnp.float32),
            pltpu.SemaphoreType.DMA((NBUF,)),
        ],
    )
    def deg(dst3_ref, zeros_ref, ones_ref, out_ref, acc, didx, ones_v, ssem):
        c = lax.axis_index("c")
        s = lax.axis_index("s")
        _per_stripe(s, lambda st, sz: pltpu.sync_copy(
            zeros_ref.at[pl.ds(0, sz)], acc.at[pl.ds(st, sz)]))
        pltpu.sync_copy(ones_ref, ones_v)
        pltpu.sync_copy(dst3_ref.at[c * NS + s], didx)
        plsc.subcore_barrier()

        def chunk(k, carry):
            pltpu.async_copy(ones_v, acc.at[didx.at[k]], ssem.at[0], add=True)
            return carry

        lax.fori_loop(0, nch, chunk, 0)

        def drain(k, carry):
            pltpu.make_async_copy(ones_v, acc.at[didx.at[k]], ssem.at[0]).wait()
            return carry

        lax.fori_loop(0, nch, drain, 0)
        plsc.subcore_barrier()

        @pl.when(c == 0)
        def _():
            _per_stripe(s, lambda st, sz: pltpu.sync_copy(
                acc.at[pl.ds(st, sz)], out_ref.at[0, pl.ds(st, sz)]))

        @pl.when(c == 1)
        def _():
            _per_stripe(s, lambda st, sz: pltpu.sync_copy(
                acc.at[pl.ds(st, sz)], out_ref.at[1, pl.ds(st, sz)]))

    return deg


@functools.cache
def _make_prop(S):
    per_tile = E // NS
    nch = per_tile // CH
    ngrp = nch // PBUF

    @functools.partial(
        pl.kernel,
        out_type=jax.ShapeDtypeStruct((S, N, 128), jnp.float32),
        mesh=_mesh(),
        scratch_types=[
            pltpu.VMEM_SHARED((N, 128), jnp.float32),
            pltpu.VMEM((PBUF, CH), jnp.int32),
            pltpu.VMEM((PBUF, CH), jnp.int32),
            pltpu.VMEM((PBUF, CH, 128), jnp.float32),
            pltpu.SemaphoreType.DMA((PBUF,)),
            pltpu.SemaphoreType.DMA((PBUF,)),
            pltpu.SemaphoreType.DMA((PBUF,)),
            pltpu.SemaphoreType.DMA((PBUF,)),
        ],
    )
    def prop(z_ref, src_ref, dst_ref, out_ref, acc, sidx, didx, rows,
             isem, dsem, gsem, ssem):
        c = lax.axis_index("c")
        s = lax.axis_index("s")
        ebase = s * per_tile
        for slab in range(S):

            @pl.when(c == (slab % NC))
            def _(slab=slab):
                _per_stripe(s, lambda st, sz: pltpu.sync_copy(
                    z_ref.at[slab, pl.ds(st, sz)], acc.at[pl.ds(st, sz)]))
                plsc.subcore_barrier()

                def group(g, carry):
                    ids, dds, gds = [], [], []
                    for b in range(PBUF):
                        off = ebase + (g * PBUF + b) * CH

                        @pl.when(g > 0)
                        def _(b=b):
                            pltpu.make_async_copy(
                                rows.at[b], acc.at[didx.at[b]], ssem.at[b]
                            ).wait()

                        ids.append(pltpu.async_copy(
                            src_ref.at[pl.ds(off, CH)], sidx.at[b], isem.at[b]))
                        dds.append(pltpu.async_copy(
                            dst_ref.at[pl.ds(off, CH)], didx.at[b], dsem.at[b]))
                    for b in range(PBUF):
                        ids[b].wait()
                        gds.append(pltpu.async_copy(
                            z_ref.at[slab].at[sidx.at[b]], rows.at[b], gsem.at[b]))
                    for b in range(PBUF):
                        dds[b].wait()
                        gds[b].wait()
                        pltpu.async_copy(
                            rows.at[b], acc.at[didx.at[b]], ssem.at[b], add=True)
                    return carry

                lax.fori_loop(0, ngrp, group, 0)
                pltpu.make_async_copy(
                    rows.at[0], acc.at[didx.at[0]], ssem.at[0]).wait()
                toff = ebase + ngrp * PBUF * CH
                pltpu.sync_copy(src_ref.at[pl.ds(toff, CH)], sidx.at[0])
                pltpu.sync_copy(dst_ref.at[pl.ds(toff, CH)], didx.at[0])
                pltpu.async_copy(
                    z_ref.at[slab].at[sidx.at[0]], rows.at[0], gsem.at[0]).wait()
                pltpu.async_copy(
                    rows.at[0], acc.at[didx.at[0]], ssem.at[0], add=True).wait()
                for b in range(1, PBUF):
                    pltpu.make_async_copy(
                        rows.at[b], acc.at[didx.at[b]], ssem.at[b]).wait()
                plsc.subcore_barrier()
                _per_stripe(s, lambda st, sz: pltpu.sync_copy(
                    acc.at[pl.ds(st, sz)], out_ref.at[slab, pl.ds(st, sz)]))

    return prop


def _dinv_block(deg_ref):
    d = deg_ref[0][:, 0:1] + deg_ref[1][:, 0:1] + 1.0
    return lax.rsqrt(d)


def _mm1_body(deg_ref, x_ref, w_ref, out_ref):
    dinv = _dinv_block(deg_ref)
    out_ref[...] = jnp.dot(
        x_ref[...], w_ref[...], preferred_element_type=jnp.float32
    ) * dinv


def _mm1(deg, x, W):
    S = W.shape[1] // 128
    return pl.pallas_call(
        _mm1_body,
        grid=(N // TN, S),
        in_specs=[
            pl.BlockSpec((NC, TN, 128), lambda n, t: (0, n, 0)),
            pl.BlockSpec((TN, x.shape[1]), lambda n, t: (n, 0)),
            pl.BlockSpec((W.shape[0], 128), lambda n, t: (0, t)),
        ],
        out_specs=pl.BlockSpec((None, TN, 128), lambda n, t: (t, n, 0)),
        out_shape=jax.ShapeDtypeStruct((S, N, 128), jnp.float32),
    )(deg, x, W)


def _make_mid(S_in):
    def body(deg_ref, agg_ref, w_ref, b_ref, out_ref):
        dinv = _dinv_block(deg_ref)
        hs = [
            jnp.maximum(agg_ref[si] * dinv + b_ref[si], 0.0) for si in range(S_in)
        ]
        h = jnp.concatenate(hs, axis=1)
        out_ref[...] = jnp.dot(
            h, w_ref[...], preferred_element_type=jnp.float32
        ) * dinv

    def run(deg, agg, W, b):
        S_out = W.shape[1] // 128
        return pl.pallas_call(
            body,
            grid=(N // TN, S_out),
            in_specs=[
                pl.BlockSpec((NC, TN, 128), lambda n, t: (0, n, 0)),
                pl.BlockSpec((S_in, TN, 128), lambda n, t: (0, n, 0)),
                pl.BlockSpec((W.shape[0], 128), lambda n, t: (0, t)),
                pl.BlockSpec((S_in, 1, 128), lambda n, t: (0, 0, 0)),
            ],
            out_specs=pl.BlockSpec((None, TN, 128), lambda n, t: (t, n, 0)),
            out_shape=jax.ShapeDtypeStruct((S_out, N, 128), jnp.float32),
        )(deg, agg, W, b.reshape(S_in, 1, 128))

    return run


_mm_mid = _make_mid(4)


def _fin_body(deg_ref, agg_ref, b_ref, h_ref, ls_ref):
    dinv = _dinv_block(deg_ref)
    h = jnp.concatenate(
        [agg_ref[0] * dinv + b_ref[0], agg_ref[1] * dinv + b_ref[1]], axis=1
    )
    h_ref[...] = h
    m = jnp.max(h, axis=1, keepdims=True)
    lse = jnp.log(jnp.sum(jnp.exp(h - m), axis=1, keepdims=True)) + m
    ls_ref[...] = h - lse


def _fin(deg, agg, b):
    return pl.pallas_call(
        _fin_body,
        grid=(N // TN,),
        in_specs=[
            pl.BlockSpec((NC, TN, 128), lambda n: (0, n, 0)),
            pl.BlockSpec((2, TN, 128), lambda n: (0, n, 0)),
            pl.BlockSpec((2, 1, 128), lambda n: (0, 0, 0)),
        ],
        out_specs=[
            pl.BlockSpec((TN, 256), lambda n: (n, 0)),
            pl.BlockSpec((TN, 256), lambda n: (n, 0)),
        ],
        out_shape=[
            jax.ShapeDtypeStruct((N, 256), jnp.float32),
            jax.ShapeDtypeStruct((N, 256), jnp.float32),
        ],
    )(deg, agg, b.reshape(2, 1, 128))


@jax.jit
def kernel(x, edge_index, W1, b1, W2, b2, W3, b3):
    ei = edge_index.astype(jnp.int32)
    src = ei[0]
    dst = ei[1]
    dst3d = dst.reshape(NC * NS, -1, DCH)
    zeros16 = jnp.zeros((LAST, 128), jnp.float32)
    ones16 = jnp.ones((DCH, 128), jnp.float32)
    deg = _make_deg()(dst3d, zeros16, ones16)

    z1 = _mm1(deg, x, W1)
    agg1 = _make_prop(4)(z1, src, dst)
    z2 = _mm_mid(deg, agg1, W2, b1)
    agg2 = _make_prop(4)(z2, src, dst)
    z3 = _mm_mid(deg, agg2, W3, b2)
    agg3 = _make_prop(2)(z3, src, dst)
    h, ls = _fin(deg, agg3, b3)
    return (h, ls)


# speedup vs baseline: 12.4538x; 1.0395x over previous
import functools
import jax
import jax.numpy as jnp
from jax import lax
from jax.experimental import pallas as pl
from jax.experimental.pallas import tpu as pltpu
from jax.experimental.pallas import tpu_sc as plsc

N = 10000
E = 160000
NC = 2
NS = 16
STRIPE = 624
LAST = N - (NS - 1) * STRIPE
CH = 80
PBUF = 4
DCH = 40
TN = 2000


def _mesh():
    return plsc.VectorSubcoreMesh(
        core_axis_name="c", subcore_axis_name="s", num_cores=NC, num_subcores=NS
    )


def _per_stripe(s, do):
    @pl.when(s < NS - 1)
    def _():
        do(s * STRIPE, STRIPE)

    @pl.when(s == NS - 1)
    def _():
        do((NS - 1) * STRIPE, LAST)


NBUF = 5

@functools.cache
def _make_deg():
    nch = E // (NC * NS) // DCH

    @functools.partial(
        pl.kernel,
        out_type=jax.ShapeDtypeStruct((NC, N, 128), jnp.float32),
        mesh=_mesh(),
        scratch_types=[
            pltpu.VMEM_SHARED((N, 128), jnp.float32),
            pltpu.VMEM((nch, DCH), jnp.int32),
            pltpu.VMEM((DCH, 128), jnp.float32),
            pltpu.SemaphoreType.DMA((NBUF,)),
        ],
    )
    def deg(dst3_ref, zeros_ref, ones_ref, out_ref, acc, didx, ones_v, ssem):
        c = lax.axis_index("c")
        s = lax.axis_index("s")
        _per_stripe(s, lambda st, sz: pltpu.sync_copy(
            zeros_ref.at[pl.ds(0, sz)], acc.at[pl.ds(st, sz)]))
        pltpu.sync_copy(ones_ref, ones_v)
        pltpu.sync_copy(dst3_ref.at[c * NS + s], didx)
        plsc.subcore_barrier()

        def chunk(k, carry):
            pltpu.async_copy(ones_v, acc.at[didx.at[k]], ssem.at[0], add=True)
            return carry

        lax.fori_loop(0, nch, chunk, 0)

        def drain(k, carry):
            pltpu.make_async_copy(ones_v, acc.at[didx.at[k]], ssem.at[0]).wait()
            return carry

        lax.fori_loop(0, nch, drain, 0)
        plsc.subcore_barrier()

        @pl.when(c == 0)
        def _():
            _per_stripe(s, lambda st, sz: pltpu.sync_copy(
                acc.at[pl.ds(st, sz)], out_ref.at[0, pl.ds(st, sz)]))

        @pl.when(c == 1)
        def _():
            _per_stripe(s, lambda st, sz: pltpu.sync_copy(
                acc.at[pl.ds(st, sz)], out_ref.at[1, pl.ds(st, sz)]))

    return deg


@functools.cache
def _make_prop(S):
    per_tile = E // NS
    nch = per_tile // CH
    ngrp = nch // PBUF

    @functools.partial(
        pl.kernel,
        out_type=jax.ShapeDtypeStruct((S, N, 128), jnp.float32),
        mesh=_mesh(),
        scratch_types=[
            pltpu.VMEM_SHARED((N, 128), jnp.float32),
            pltpu.VMEM((PBUF, CH), jnp.int32),
            pltpu.VMEM((PBUF, CH), jnp.int32),
            pltpu.VMEM((PBUF, CH, 128), jnp.float32),
            pltpu.SemaphoreType.DMA((PBUF,)),
            pltpu.SemaphoreType.DMA((PBUF,)),
            pltpu.SemaphoreType.DMA((PBUF,)),
            pltpu.SemaphoreType.DMA((PBUF,)),
        ],
    )
    def prop(z_ref, src_ref, dst_ref, out_ref, acc, sidx, didx, rows,
             isem, dsem, gsem, ssem):
        c = lax.axis_index("c")
        s = lax.axis_index("s")
        ebase = s * per_tile
        for slab in range(S):

            @pl.when(c == (slab % NC))
            def _(slab=slab):
                _per_stripe(s, lambda st, sz: pltpu.sync_copy(
                    z_ref.at[slab, pl.ds(st, sz)], acc.at[pl.ds(st, sz)]))
                plsc.subcore_barrier()

                def group(g, carry):
                    ids, dds, gds = [], [], []
                    for b in range(PBUF):
                        off = ebase + (g * PBUF + b) * CH

                        @pl.when(g > 0)
                        def _(b=b):
                            pltpu.make_async_copy(
                                rows.at[b], acc.at[didx.at[b]], ssem.at[b]
                            ).wait()

                        ids.append(pltpu.async_copy(
                            src_ref.at[pl.ds(off, CH)], sidx.at[b], isem.at[b]))
                        dds.append(pltpu.async_copy(
                            dst_ref.at[pl.ds(off, CH)], didx.at[b], dsem.at[b]))
                    for b in range(PBUF):
                        ids[b].wait()
                        gds.append(pltpu.async_copy(
                            z_ref.at[slab].at[sidx.at[b]], rows.at[b], gsem.at[b]))
                    for b in range(PBUF):
                        dds[b].wait()
                        gds[b].wait()
                        pltpu.async_copy(
                            rows.at[b], acc.at[didx.at[b]], ssem.at[b], add=True)
                    return carry

                lax.fori_loop(0, ngrp, group, 0)
                pltpu.make_async_copy(
                    rows.at[0], acc.at[didx.at[0]], ssem.at[0]).wait()
                toff = ebase + ngrp * PBUF * CH
                pltpu.sync_copy(src_ref.at[pl.ds(toff, CH)], sidx.at[0])
                pltpu.sync_copy(dst_ref.at[pl.ds(toff, CH)], didx.at[0])
                pltpu.async_copy(
                    z_ref.at[slab].at[sidx.at[0]], rows.at[0], gsem.at[0]).wait()
                pltpu.async_copy(
                    rows.at[0], acc.at[didx.at[0]], ssem.at[0], add=True).wait()
                for b in range(1, PBUF):
                    pltpu.make_async_copy(
                        rows.at[b], acc.at[didx.at[b]], ssem.at[b]).wait()
                plsc.subcore_barrier()
                _per_stripe(s, lambda st, sz: pltpu.sync_copy(
                    acc.at[pl.ds(st, sz)], out_ref.at[slab, pl.ds(st, sz)]))

    return prop


def _dinv_block(deg_ref):
    d = deg_ref[0][:, 0:1] + deg_ref[1][:, 0:1] + 1.0
    return lax.rsqrt(d)


def _mm1_body(deg_ref, x_ref, w_ref, out_ref):
    dinv = _dinv_block(deg_ref)
    out_ref[...] = jnp.dot(
        x_ref[...], w_ref[...], preferred_element_type=jnp.float32
    ) * dinv


def _mm1(deg, x, W):
    S = W.shape[1] // 128
    return pl.pallas_call(
        _mm1_body,
        grid=(N // TN, S),
        in_specs=[
            pl.BlockSpec((NC, TN, 128), lambda n, t: (0, n, 0)),
            pl.BlockSpec((TN, x.shape[1]), lambda n, t: (n, 0)),
            pl.BlockSpec((W.shape[0], 128), lambda n, t: (0, t)),
        ],
        out_specs=pl.BlockSpec((None, TN, 128), lambda n, t: (t, n, 0)),
        out_shape=jax.ShapeDtypeStruct((S, N, 128), jnp.float32),
    )(deg, x, W)


def _make_mid(S_in):
    def body(deg_ref, agg_ref, w_ref, b_ref, out_ref):
        dinv = _dinv_block(deg_ref)
        hs = [
            jnp.maximum(agg_ref[si] * dinv + b_ref[si], 0.0) for si in range(S_in)
        ]
        h = jnp.concatenate(hs, axis=1)
        out_ref[...] = jnp.dot(
            h, w_ref[...], preferred_element_type=jnp.float32
        ) * dinv

    def run(deg, agg, W, b):
        S_out = W.shape[1] // 128
        return pl.pallas_call(
            body,
            grid=(N // TN, S_out),
            in_specs=[
                pl.BlockSpec((NC, TN, 128), lambda n, t: (0, n, 0)),
                pl.BlockSpec((S_in, TN, 128), lambda n, t: (0, n, 0)),
                pl.BlockSpec((W.shape[0], 128), lambda n, t: (0, t)),
                pl.BlockSpec((S_in, 1, 128), lambda n, t: (0, 0, 0)),
            ],
            out_specs=pl.BlockSpec((None, TN, 128), lambda n, t: (t, n, 0)),
            out_shape=jax.ShapeDtypeStruct((S_out, N, 128), jnp.float32),
        )(deg, agg, W, b.reshape(S_in, 1, 128))

    return run


_mm_mid = _make_mid(4)


def _fin_body(deg_ref, agg_ref, b_ref, h_ref, ls_ref):
    dinv = _dinv_block(deg_ref)
    h = jnp.concatenate(
        [agg_ref[0] * dinv + b_ref[0], agg_ref[1] * dinv + b_ref[1]], axis=1
    )
    h_ref[...] = h
    m = jnp.max(h, axis=1, keepdims=True)
    lse = jnp.log(jnp.sum(jnp.exp(h - m), axis=1, keepdims=True)) + m
    ls_ref[...] = h - lse


def _fin(deg, agg, b):
    return pl.pallas_call(
        _fin_body,
        grid=(N // TN,),
        in_specs=[
            pl.BlockSpec((NC, TN, 128), lambda n: (0, n, 0)),
            pl.BlockSpec((2, TN, 128), lambda n: (0, n, 0)),
            pl.BlockSpec((2, 1, 128), lambda n: (0, 0, 0)),
        ],
        out_specs=[
            pl.BlockSpec((TN, 256), lambda n: (n, 0)),
            pl.BlockSpec((TN, 256), lambda n: (n, 0)),
        ],
        out_shape=[
            jax.ShapeDtypeStruct((N, 256), jnp.float32),
            jax.ShapeDtypeStruct((N, 256), jnp.float32),
        ],
    )(deg, agg, b.reshape(2, 1, 128))


@jax.jit
def kernel(x, edge_index, W1, b1, W2, b2, W3, b3):
    ei = edge_index.astype(jnp.int32)
    src = ei[0]
    dst = ei[1]
    dst3d = dst.reshape(NC * NS, -1, DCH)
    zeros16 = jnp.zeros((LAST, 128), jnp.float32)
    ones16 = jnp.ones((DCH, 128), jnp.float32)
    deg = _make_deg()(dst3d, zeros16, ones16)

    z1 = _mm1(deg, x, W1)
    agg1 = _make_prop(4)(z1, src, dst)
    z2 = _mm_mid(deg, agg1, W2, b1)
    agg2 = _make_prop(4)(z2, src, dst)
    z3 = _mm_mid(deg, agg2, W3, b2)
    agg3 = _make_prop(2)(z3, src, dst)
    h, ls = _fin(deg, agg3, b3)
    return (h, ls)


# speedup vs baseline: 12.7569x; 1.0243x over previous
import functools
import jax
import jax.numpy as jnp
from jax import lax
from jax.experimental import pallas as pl
from jax.experimental.pallas import tpu as pltpu
from jax.experimental.pallas import tpu_sc as plsc

N = 10000
E = 160000
NC = 2
NS = 16
STRIPE = 624
LAST = N - (NS - 1) * STRIPE
CH = 80
PBUF = 4
DCH = 40
TN = 5000


def _mesh():
    return plsc.VectorSubcoreMesh(
        core_axis_name="c", subcore_axis_name="s", num_cores=NC, num_subcores=NS
    )


def _per_stripe(s, do):
    @pl.when(s < NS - 1)
    def _():
        do(s * STRIPE, STRIPE)

    @pl.when(s == NS - 1)
    def _():
        do((NS - 1) * STRIPE, LAST)


NBUF = 5

@functools.cache
def _make_deg():
    nch = E // (NC * NS) // DCH

    @functools.partial(
        pl.kernel,
        out_type=jax.ShapeDtypeStruct((NC, N, 128), jnp.float32),
        mesh=_mesh(),
        scratch_types=[
            pltpu.VMEM_SHARED((N, 128), jnp.float32),
            pltpu.VMEM((nch, DCH), jnp.int32),
            pltpu.VMEM((DCH, 128), jnp.float32),
            pltpu.SemaphoreType.DMA((NBUF,)),
        ],
    )
    def deg(dst3_ref, zeros_ref, ones_ref, out_ref, acc, didx, ones_v, ssem):
        c = lax.axis_index("c")
        s = lax.axis_index("s")
        _per_stripe(s, lambda st, sz: pltpu.sync_copy(
            zeros_ref.at[pl.ds(0, sz)], acc.at[pl.ds(st, sz)]))
        pltpu.sync_copy(ones_ref, ones_v)
        pltpu.sync_copy(dst3_ref.at[c * NS + s], didx)
        plsc.subcore_barrier()

        def chunk(k, carry):
            pltpu.async_copy(ones_v, acc.at[didx.at[k]], ssem.at[0], add=True)
            return carry

        lax.fori_loop(0, nch, chunk, 0)

        def drain(k, carry):
            pltpu.make_async_copy(ones_v, acc.at[didx.at[k]], ssem.at[0]).wait()
            return carry

        lax.fori_loop(0, nch, drain, 0)
        plsc.subcore_barrier()

        @pl.when(c == 0)
        def _():
            _per_stripe(s, lambda st, sz: pltpu.sync_copy(
                acc.at[pl.ds(st, sz)], out_ref.at[0, pl.ds(st, sz)]))

        @pl.when(c == 1)
        def _():
            _per_stripe(s, lambda st, sz: pltpu.sync_copy(
                acc.at[pl.ds(st, sz)], out_ref.at[1, pl.ds(st, sz)]))

    return deg


@functools.cache
def _make_prop(S):
    per_tile = E // NS
    nch = per_tile // CH
    ngrp = nch // PBUF

    @functools.partial(
        pl.kernel,
        out_type=jax.ShapeDtypeStruct((S, N, 128), jnp.float32),
        mesh=_mesh(),
        scratch_types=[
            pltpu.VMEM_SHARED((N, 128), jnp.float32),
            pltpu.VMEM((PBUF, CH), jnp.int32),
            pltpu.VMEM((PBUF, CH), jnp.int32),
            pltpu.VMEM((PBUF, CH, 128), jnp.float32),
            pltpu.SemaphoreType.DMA((PBUF,)),
            pltpu.SemaphoreType.DMA((PBUF,)),
            pltpu.SemaphoreType.DMA((PBUF,)),
            pltpu.SemaphoreType.DMA((PBUF,)),
        ],
    )
    def prop(z_ref, src_ref, dst_ref, out_ref, acc, sidx, didx, rows,
             isem, dsem, gsem, ssem):
        c = lax.axis_index("c")
        s = lax.axis_index("s")
        ebase = s * per_tile
        for slab in range(S):

            @pl.when(c == (slab % NC))
            def _(slab=slab):
                _per_stripe(s, lambda st, sz: pltpu.sync_copy(
                    z_ref.at[slab, pl.ds(st, sz)], acc.at[pl.ds(st, sz)]))
                plsc.subcore_barrier()

                def group(g, carry):
                    ids, dds, gds = [], [], []
                    for b in range(PBUF):
                        off = ebase + (g * PBUF + b) * CH

                        @pl.when(g > 0)
                        def _(b=b):
                            pltpu.make_async_copy(
                                rows.at[b], acc.at[didx.at[b]], ssem.at[b]
                            ).wait()

                        ids.append(pltpu.async_copy(
                            src_ref.at[pl.ds(off, CH)], sidx.at[b], isem.at[b]))
                        dds.append(pltpu.async_copy(
                            dst_ref.at[pl.ds(off, CH)], didx.at[b], dsem.at[b]))
                    for b in range(PBUF):
                        ids[b].wait()
                        gds.append(pltpu.async_copy(
                            z_ref.at[slab].at[sidx.at[b]], rows.at[b], gsem.at[b]))
                    for b in range(PBUF):
                        dds[b].wait()
                        gds[b].wait()
                        pltpu.async_copy(
                            rows.at[b], acc.at[didx.at[b]], ssem.at[b], add=True)
                    return carry

                lax.fori_loop(0, ngrp, group, 0)
                pltpu.make_async_copy(
                    rows.at[0], acc.at[didx.at[0]], ssem.at[0]).wait()
                toff = ebase + ngrp * PBUF * CH
                pltpu.sync_copy(src_ref.at[pl.ds(toff, CH)], sidx.at[0])
                pltpu.sync_copy(dst_ref.at[pl.ds(toff, CH)], didx.at[0])
                pltpu.async_copy(
                    z_ref.at[slab].at[sidx.at[0]], rows.at[0], gsem.at[0]).wait()
                pltpu.async_copy(
                    rows.at[0], acc.at[didx.at[0]], ssem.at[0], add=True).wait()
                for b in range(1, PBUF):
                    pltpu.make_async_copy(
                        rows.at[b], acc.at[didx.at[b]], ssem.at[b]).wait()
                plsc.subcore_barrier()
                _per_stripe(s, lambda st, sz: pltpu.sync_copy(
                    acc.at[pl.ds(st, sz)], out_ref.at[slab, pl.ds(st, sz)]))

    return prop


def _dinv_block(deg_ref):
    d = deg_ref[0][:, 0:1] + deg_ref[1][:, 0:1] + 1.0
    return lax.rsqrt(d)


def _mm1_body(deg_ref, x_ref, w_ref, out_ref):
    dinv = _dinv_block(deg_ref)
    out_ref[...] = jnp.dot(
        x_ref[...], w_ref[...], preferred_element_type=jnp.float32
    ) * dinv


def _mm1(deg, x, W):
    S = W.shape[1] // 128
    return pl.pallas_call(
        _mm1_body,
        grid=(N // TN, S),
        in_specs=[
            pl.BlockSpec((NC, TN, 128), lambda n, t: (0, n, 0)),
            pl.BlockSpec((TN, x.shape[1]), lambda n, t: (n, 0)),
            pl.BlockSpec((W.shape[0], 128), lambda n, t: (0, t)),
        ],
        out_specs=pl.BlockSpec((None, TN, 128), lambda n, t: (t, n, 0)),
        out_shape=jax.ShapeDtypeStruct((S, N, 128), jnp.float32),
    )(deg, x, W)


def _make_mid(S_in):
    def body(deg_ref, agg_ref, w_ref, b_ref, out_ref):
        dinv = _dinv_block(deg_ref)
        hs = [
            jnp.maximum(agg_ref[si] * dinv + b_ref[si], 0.0) for si in range(S_in)
        ]
        h = jnp.concatenate(hs, axis=1)
        out_ref[...] = jnp.dot(
            h, w_ref[...], preferred_element_type=jnp.float32
        ) * dinv

    def run(deg, agg, W, b):
        S_out = W.shape[1] // 128
        return pl.pallas_call(
            body,
            grid=(N // TN, S_out),
            in_specs=[
                pl.BlockSpec((NC, TN, 128), lambda n, t: (0, n, 0)),
                pl.BlockSpec((S_in, TN, 128), lambda n, t: (0, n, 0)),
                pl.BlockSpec((W.shape[0], 128), lambda n, t: (0, t)),
                pl.BlockSpec((S_in, 1, 128), lambda n, t: (0, 0, 0)),
            ],
            out_specs=pl.BlockSpec((None, TN, 128), lambda n, t: (t, n, 0)),
            out_shape=jax.ShapeDtypeStruct((S_out, N, 128), jnp.float32),
        )(deg, agg, W, b.reshape(S_in, 1, 128))

    return run


_mm_mid = _make_mid(4)


def _fin_body(deg_ref, agg_ref, b_ref, h_ref, ls_ref):
    dinv = _dinv_block(deg_ref)
    h = jnp.concatenate(
        [agg_ref[0] * dinv + b_ref[0], agg_ref[1] * dinv + b_ref[1]], axis=1
    )
    h_ref[...] = h
    m = jnp.max(h, axis=1, keepdims=True)
    lse = jnp.log(jnp.sum(jnp.exp(h - m), axis=1, keepdims=True)) + m
    ls_ref[...] = h - lse


def _fin(deg, agg, b):
    return pl.pallas_call(
        _fin_body,
        grid=(N // TN,),
        in_specs=[
            pl.BlockSpec((NC, TN, 128), lambda n: (0, n, 0)),
            pl.BlockSpec((2, TN, 128), lambda n: (0, n, 0)),
            pl.BlockSpec((2, 1, 128), lambda n: (0, 0, 0)),
        ],
        out_specs=[
            pl.BlockSpec((TN, 256), lambda n: (n, 0)),
            pl.BlockSpec((TN, 256), lambda n: (n, 0)),
        ],
        out_shape=[
            jax.ShapeDtypeStruct((N, 256), jnp.float32),
            jax.ShapeDtypeStruct((N, 256), jnp.float32),
        ],
    )(deg, agg, b.reshape(2, 1, 128))


@jax.jit
def kernel(x, edge_index, W1, b1, W2, b2, W3, b3):
    ei = edge_index.astype(jnp.int32)
    src = ei[0]
    dst = ei[1]
    dst3d = dst.reshape(NC * NS, -1, DCH)
    zeros16 = jnp.zeros((LAST, 128), jnp.float32)
    ones16 = jnp.ones((DCH, 128), jnp.float32)
    deg = _make_deg()(dst3d, zeros16, ones16)

    z1 = _mm1(deg, x, W1)
    agg1 = _make_prop(4)(z1, src, dst)
    z2 = _mm_mid(deg, agg1, W2, b1)
    agg2 = _make_prop(4)(z2, src, dst)
    z3 = _mm_mid(deg, agg2, W3, b2)
    agg3 = _make_prop(2)(z3, src, dst)
    h, ls = _fin(deg, agg3, b3)
    return (h, ls)


# speedup vs baseline: 12.8143x; 1.0045x over previous
import functools
import jax
import jax.numpy as jnp
from jax import lax
from jax.experimental import pallas as pl
from jax.experimental.pallas import tpu as pltpu
from jax.experimental.pallas import tpu_sc as plsc

N = 10000
E = 160000
NC = 2
NS = 16
STRIPE = 624
LAST = N - (NS - 1) * STRIPE
CH = 80
PBUF = 4
DCH = 40
TN = 10000


def _mesh():
    return plsc.VectorSubcoreMesh(
        core_axis_name="c", subcore_axis_name="s", num_cores=NC, num_subcores=NS
    )


def _per_stripe(s, do):
    @pl.when(s < NS - 1)
    def _():
        do(s * STRIPE, STRIPE)

    @pl.when(s == NS - 1)
    def _():
        do((NS - 1) * STRIPE, LAST)


NBUF = 5

@functools.cache
def _make_deg():
    nch = E // (NC * NS) // DCH

    @functools.partial(
        pl.kernel,
        out_type=jax.ShapeDtypeStruct((NC, N, 128), jnp.float32),
        mesh=_mesh(),
        scratch_types=[
            pltpu.VMEM_SHARED((N, 128), jnp.float32),
            pltpu.VMEM((nch, DCH), jnp.int32),
            pltpu.VMEM((DCH, 128), jnp.float32),
            pltpu.SemaphoreType.DMA((NBUF,)),
        ],
    )
    def deg(dst3_ref, zeros_ref, ones_ref, out_ref, acc, didx, ones_v, ssem):
        c = lax.axis_index("c")
        s = lax.axis_index("s")
        _per_stripe(s, lambda st, sz: pltpu.sync_copy(
            zeros_ref.at[pl.ds(0, sz)], acc.at[pl.ds(st, sz)]))
        pltpu.sync_copy(ones_ref, ones_v)
        pltpu.sync_copy(dst3_ref.at[c * NS + s], didx)
        plsc.subcore_barrier()

        def chunk(k, carry):
            pltpu.async_copy(ones_v, acc.at[didx.at[k]], ssem.at[0], add=True)
            return carry

        lax.fori_loop(0, nch, chunk, 0)

        def drain(k, carry):
            pltpu.make_async_copy(ones_v, acc.at[didx.at[k]], ssem.at[0]).wait()
            return carry

        lax.fori_loop(0, nch, drain, 0)
        plsc.subcore_barrier()

        @pl.when(c == 0)
        def _():
            _per_stripe(s, lambda st, sz: pltpu.sync_copy(
                acc.at[pl.ds(st, sz)], out_ref.at[0, pl.ds(st, sz)]))

        @pl.when(c == 1)
        def _():
            _per_stripe(s, lambda st, sz: pltpu.sync_copy(
                acc.at[pl.ds(st, sz)], out_ref.at[1, pl.ds(st, sz)]))

    return deg


@functools.cache
def _make_prop(S):
    per_tile = E // NS
    nch = per_tile // CH
    ngrp = nch // PBUF

    @functools.partial(
        pl.kernel,
        out_type=jax.ShapeDtypeStruct((S, N, 128), jnp.float32),
        mesh=_mesh(),
        scratch_types=[
            pltpu.VMEM_SHARED((N, 128), jnp.float32),
            pltpu.VMEM((PBUF, CH), jnp.int32),
            pltpu.VMEM((PBUF, CH), jnp.int32),
            pltpu.VMEM((PBUF, CH, 128), jnp.float32),
            pltpu.SemaphoreType.DMA((PBUF,)),
            pltpu.SemaphoreType.DMA((PBUF,)),
            pltpu.SemaphoreType.DMA((PBUF,)),
            pltpu.SemaphoreType.DMA((PBUF,)),
        ],
    )
    def prop(z_ref, src_ref, dst_ref, out_ref, acc, sidx, didx, rows,
             isem, dsem, gsem, ssem):
        c = lax.axis_index("c")
        s = lax.axis_index("s")
        ebase = s * per_tile
        for slab in range(S):

            @pl.when(c == (slab % NC))
            def _(slab=slab):
                _per_stripe(s, lambda st, sz: pltpu.sync_copy(
                    z_ref.at[slab, pl.ds(st, sz)], acc.at[pl.ds(st, sz)]))
                plsc.subcore_barrier()

                def group(g, carry):
                    ids, dds, gds = [], [], []
                    for b in range(PBUF):
                        off = ebase + (g * PBUF + b) * CH

                        @pl.when(g > 0)
                        def _(b=b):
                            pltpu.make_async_copy(
                                rows.at[b], acc.at[didx.at[b]], ssem.at[b]
                            ).wait()

                        ids.append(pltpu.async_copy(
                            src_ref.at[pl.ds(off, CH)], sidx.at[b], isem.at[b]))
                        dds.append(pltpu.async_copy(
                            dst_ref.at[pl.ds(off, CH)], didx.at[b], dsem.at[b]))
                    for b in range(PBUF):
                        ids[b].wait()
                        gds.append(pltpu.async_copy(
                            z_ref.at[slab].at[sidx.at[b]], rows.at[b], gsem.at[b]))
                    for b in range(PBUF):
                        dds[b].wait()
                        gds[b].wait()
                        pltpu.async_copy(
                            rows.at[b], acc.at[didx.at[b]], ssem.at[b], add=True)
                    return carry

                lax.fori_loop(0, ngrp, group, 0)
                pltpu.make_async_copy(
                    rows.at[0], acc.at[didx.at[0]], ssem.at[0]).wait()
                toff = ebase + ngrp * PBUF * CH
                pltpu.sync_copy(src_ref.at[pl.ds(toff, CH)], sidx.at[0])
                pltpu.sync_copy(dst_ref.at[pl.ds(toff, CH)], didx.at[0])
                pltpu.async_copy(
                    z_ref.at[slab].at[sidx.at[0]], rows.at[0], gsem.at[0]).wait()
                pltpu.async_copy(
                    rows.at[0], acc.at[didx.at[0]], ssem.at[0], add=True).wait()
                for b in range(1, PBUF):
                    pltpu.make_async_copy(
                        rows.at[b], acc.at[didx.at[b]], ssem.at[b]).wait()
                plsc.subcore_barrier()
                _per_stripe(s, lambda st, sz: pltpu.sync_copy(
                    acc.at[pl.ds(st, sz)], out_ref.at[slab, pl.ds(st, sz)]))

    return prop


def _dinv_block(deg_ref):
    d = deg_ref[0][:, 0:1] + deg_ref[1][:, 0:1] + 1.0
    return lax.rsqrt(d)


def _mm1_body(deg_ref, x_ref, w_ref, out_ref):
    dinv = _dinv_block(deg_ref)
    out_ref[...] = jnp.dot(
        x_ref[...], w_ref[...], preferred_element_type=jnp.float32
    ) * dinv


def _mm1(deg, x, W):
    S = W.shape[1] // 128
    return pl.pallas_call(
        _mm1_body,
        grid=(N // TN, S),
        in_specs=[
            pl.BlockSpec((NC, TN, 128), lambda n, t: (0, n, 0)),
            pl.BlockSpec((TN, x.shape[1]), lambda n, t: (n, 0)),
            pl.BlockSpec((W.shape[0], 128), lambda n, t: (0, t)),
        ],
        out_specs=pl.BlockSpec((None, TN, 128), lambda n, t: (t, n, 0)),
        out_shape=jax.ShapeDtypeStruct((S, N, 128), jnp.float32),
    )(deg, x, W)


def _make_mid(S_in):
    def body(deg_ref, agg_ref, w_ref, b_ref, out_ref):
        dinv = _dinv_block(deg_ref)
        hs = [
            jnp.maximum(agg_ref[si] * dinv + b_ref[si], 0.0) for si in range(S_in)
        ]
        h = jnp.concatenate(hs, axis=1)
        out_ref[...] = jnp.dot(
            h, w_ref[...], preferred_element_type=jnp.float32
        ) * dinv

    def run(deg, agg, W, b):
        S_out = W.shape[1] // 128
        return pl.pallas_call(
            body,
            grid=(N // TN, S_out),
            in_specs=[
                pl.BlockSpec((NC, TN, 128), lambda n, t: (0, n, 0)),
                pl.BlockSpec((S_in, TN, 128), lambda n, t: (0, n, 0)),
                pl.BlockSpec((W.shape[0], 128), lambda n, t: (0, t)),
                pl.BlockSpec((S_in, 1, 128), lambda n, t: (0, 0, 0)),
            ],
            out_specs=pl.BlockSpec((None, TN, 128), lambda n, t: (t, n, 0)),
            out_shape=jax.ShapeDtypeStruct((S_out, N, 128), jnp.float32),
        )(deg, agg, W, b.reshape(S_in, 1, 128))

    return run


_mm_mid = _make_mid(4)


def _fin_body(deg_ref, agg_ref, b_ref, h_ref, ls_ref):
    dinv = _dinv_block(deg_ref)
    h = jnp.concatenate(
        [agg_ref[0] * dinv + b_ref[0], agg_ref[1] * dinv + b_ref[1]], axis=1
    )
    h_ref[...] = h
    m = jnp.max(h, axis=1, keepdims=True)
    lse = jnp.log(jnp.sum(jnp.exp(h - m), axis=1, keepdims=True)) + m
    ls_ref[...] = h - lse


def _fin(deg, agg, b):
    return pl.pallas_call(
        _fin_body,
        grid=(N // TN,),
        in_specs=[
            pl.BlockSpec((NC, TN, 128), lambda n: (0, n, 0)),
            pl.BlockSpec((2, TN, 128), lambda n: (0, n, 0)),
            pl.BlockSpec((2, 1, 128), lambda n: (0, 0, 0)),
        ],
        out_specs=[
            pl.BlockSpec((TN, 256), lambda n: (n, 0)),
            pl.BlockSpec((TN, 256), lambda n: (n, 0)),
        ],
        out_shape=[
            jax.ShapeDtypeStruct((N, 256), jnp.float32),
            jax.ShapeDtypeStruct((N, 256), jnp.float32),
        ],
    )(deg, agg, b.reshape(2, 1, 128))


@jax.jit
def kernel(x, edge_index, W1, b1, W2, b2, W3, b3):
    ei = edge_index.astype(jnp.int32)
    src = ei[0]
    dst = ei[1]
    dst3d = dst.reshape(NC * NS, -1, DCH)
    zeros16 = jnp.zeros((LAST, 128), jnp.float32)
    ones16 = jnp.ones((DCH, 128), jnp.float32)
    deg = _make_deg()(dst3d, zeros16, ones16)

    z1 = _mm1(deg, x, W1)
    agg1 = _make_prop(4)(z1, src, dst)
    z2 = _mm_mid(deg, agg1, W2, b1)
    agg2 = _make_prop(4)(z2, src, dst)
    z3 = _mm_mid(deg, agg2, W3, b2)
    agg3 = _make_prop(2)(z3, src, dst)
    h, ls = _fin(deg, agg3, b3)
    return (h, ls)
